# baseline (speedup 1.0000x reference)
import jax
import jax.numpy as jnp
from jax import lax
from jax.experimental import pallas as pl
from jax.experimental.pallas import tpu as pltpu
from jax.experimental.pallas import tpu_sc as plsc

_NC = 2
_NS = 16
_NW = _NC * _NS
_L = 16

_B = 4096
_S = 200
_D = 64
_C = 2
_BPW = _B // _NW
_CHUNK = _S // 2
_PAIRS = _BPW // 2
_G = 16
_NGRP = _S // _G
_TAIL = _S - _NGRP * _G
_SPAD = _NGRP * _G + _G


def _tec_body(table, idx_hbm, par_hbm, w_hbm, b_hbm, out_hbm,
              idx_v, par_v, buf_a, buf_b, w_v, b_v, log_v, sem_a, sem_b):
    wid = lax.axis_index("s") * _NC + lax.axis_index("c")
    base = wid * _BPW

    pltpu.sync_copy(idx_hbm.at[pl.ds(base, _BPW)], idx_v)
    pltpu.sync_copy(par_hbm.at[pl.ds(base, _BPW)], par_v)
    pltpu.sync_copy(w_hbm, w_v)
    pltpu.sync_copy(b_hbm, b_v)

    inv_s = 1.0 / _S
    w = [[w_v[pl.ds(c * _D + k * _L, _L)] * inv_s for k in range(4)]
         for c in range(_C)]
    bvec = b_v[...]
    b0 = bvec[0]
    b1 = bvec[1]
    lane = jax.lax.iota(jnp.int32, _L)

    def issue(row, buf, sem):
        pltpu.async_copy(table.at[idx_v.at[row, 0]], buf.at[pl.ds(0, _CHUNK)], sem)
        pltpu.async_copy(table.at[idx_v.at[row, 1]], buf.at[pl.ds(_CHUNK, _CHUNK)], sem)

    def wait(buf, sem):
        pltpu.make_async_copy(table.at[pl.ds(0, _S)], buf, sem).wait()

    def acc_token(buf, r, pv, t, accs):
        sel = pv[t] > 0
        return tuple(
            accs[k]
            + jnp.where(sel,
                        buf[r, pl.ds(_D + k * _L, _L)],
                        buf[r, pl.ds(k * _L, _L)])
            for k in range(4)
        )

    def process(row, buf, lvec):
        zero = jnp.zeros((_L,), jnp.float32)

        def group_body(g, accs):
            r0 = g * _G
            pv = par_v[row, pl.ds(r0, _G)]
            for t in range(_G):
                accs = acc_token(buf, r0 + t, pv, t, accs)
            return accs

        a = lax.fori_loop(0, _NGRP, group_body, (zero, zero, zero, zero))
        pv_tail = par_v[row, pl.ds(_NGRP * _G, _G)]
        for t in range(_TAIL):
            a = acc_token(buf, _NGRP * _G + t, pv_tail, t, a)

        t0 = a[0] * w[0][0] + a[1] * w[0][1] + a[2] * w[0][2] + a[3] * w[0][3]
        t1 = a[0] * w[1][0] + a[1] * w[1][1] + a[2] * w[1][2] + a[3] * w[1][3]
        l0 = jnp.sum(t0) + b0
        l1 = jnp.sum(t1) + b1
        slot = 2 * lax.rem(row, 8)
        lvec = jnp.where(lane == slot, l0, lvec)
        lvec = jnp.where(lane == slot + 1, l1, lvec)
        return lvec

    issue(0, buf_a, sem_a)

    def outer(i, lvec):
        issue(2 * i + 1, buf_b, sem_b)
        wait(buf_a, sem_a)
        lvec = process(2 * i, buf_a, lvec)

        @pl.when(i < _PAIRS - 1)
        def _():
            issue(2 * i + 2, buf_a, sem_a)

        wait(buf_b, sem_b)
        lvec = process(2 * i + 1, buf_b, lvec)

        @pl.when(lax.rem(i, 4) == 3)
        def _():
            log_v[pl.ds((i // 4) * _L, _L)] = lvec

        return lvec

    lax.fori_loop(0, _PAIRS, outer, jnp.zeros((_L,), jnp.float32))
    pltpu.sync_copy(log_v, out_hbm.at[pl.ds(base * _C, _BPW * _C)])


@jax.jit
def _sc_call(table2, idx3, par2, w_flat, b_pad):
    mesh = plsc.VectorSubcoreMesh(core_axis_name="c", subcore_axis_name="s",
                                  num_cores=_NC, num_subcores=_NS)
    return pl.kernel(
        _tec_body,
        out_type=jax.ShapeDtypeStruct((_B * _C,), jnp.float32),
        mesh=mesh,
        compiler_params=pltpu.CompilerParams(needs_layout_passes=False,
                                             use_tc_tiling_on_sc=True),
        scratch_types=[
            pltpu.VMEM((_BPW, 2, _CHUNK), jnp.int32),
            pltpu.VMEM((_BPW, _SPAD), jnp.int32),
            pltpu.VMEM((_S, 2 * _D), jnp.float32),
            pltpu.VMEM((_S, 2 * _D), jnp.float32),
            pltpu.VMEM((_C * _D,), jnp.float32),
            pltpu.VMEM((_L,), jnp.float32),
            pltpu.VMEM((_BPW * _C,), jnp.float32),
            pltpu.SemaphoreType.DMA,
            pltpu.SemaphoreType.DMA,
        ],
    )(table2, idx3, par2, w_flat, b_pad)


def kernel(x, embedding, W, b):
    xi = x.astype(jnp.int32)
    table2 = embedding.reshape(-1, 2 * _D)
    idx3 = (xi >> 1).reshape(_B, 2, _CHUNK)
    par2 = jnp.pad(xi & 1, ((0, 0), (0, _SPAD - _S))).reshape(_B, _SPAD)
    w_flat = W.astype(jnp.float32).reshape(-1)
    b_pad = jnp.pad(b.astype(jnp.float32), (0, _L - _C))
    return _sc_call(table2, idx3, par2, w_flat, b_pad).reshape(_B, _C)


# speedup vs baseline: 1.3275x; 1.3275x over previous
import jax
import jax.numpy as jnp
from jax import lax
from jax.experimental import pallas as pl
from jax.experimental.pallas import tpu as pltpu
from jax.experimental.pallas import tpu_sc as plsc

_NC = 2
_NS = 16
_NW = _NC * _NS
_L = 16

_B = 4096
_S = 200
_D = 64
_C = 2
_BPW = _B // _NW
_CHUNK = _S // 2
_PAIRS = _BPW // 2
_V = 1000000
_TBLK = 1024
_NTBLK = (_V + 2 * _TBLK - 1) // (2 * _TBLK)
_VSUP = _NTBLK * _TBLK
_G = 16
_NGRP = _S // _G
_TAIL = _S - _NGRP * _G
_SPAD = _NGRP * _G + _G


def _transpose_body(in_ref, out_ref):
    x = in_ref[...]
    x2 = jnp.concatenate([x[:, 0:_TBLK], x[:, _TBLK:2 * _TBLK]], axis=0)
    out_ref[...] = jnp.transpose(x2, (1, 0))


def _tec_body(table, idx_hbm, par_hbm, w_hbm, b_hbm, out_hbm,
              idx_v, par_v, buf_a, buf_b, w_v, b_v, log_v, sem_a, sem_b):
    wid = lax.axis_index("s") * _NC + lax.axis_index("c")
    base = wid * _BPW

    pltpu.sync_copy(idx_hbm.at[pl.ds(base, _BPW)], idx_v)
    pltpu.sync_copy(par_hbm.at[pl.ds(base, _BPW)], par_v)
    pltpu.sync_copy(w_hbm, w_v)
    pltpu.sync_copy(b_hbm, b_v)

    inv_s = 1.0 / _S
    w = [[w_v[pl.ds(c * _D + k * _L, _L)] * inv_s for k in range(4)]
         for c in range(_C)]
    bvec = b_v[...]
    b0 = bvec[0]
    b1 = bvec[1]
    lane = jax.lax.iota(jnp.int32, _L)

    def issue(row, buf, sem):
        pltpu.async_copy(table.at[idx_v.at[row, 0]], buf.at[pl.ds(0, _CHUNK)], sem)
        pltpu.async_copy(table.at[idx_v.at[row, 1]], buf.at[pl.ds(_CHUNK, _CHUNK)], sem)

    def wait(buf, sem):
        pltpu.make_async_copy(table.at[pl.ds(0, _S)], buf, sem).wait()

    def acc_token(buf, r, pv, t, accs):
        sel = pv[t] > 0
        return tuple(
            accs[k]
            + jnp.where(sel,
                        buf[r, pl.ds(_D + k * _L, _L)],
                        buf[r, pl.ds(k * _L, _L)])
            for k in range(4)
        )

    def process(row, buf, lvec):
        zero = jnp.zeros((_L,), jnp.float32)

        def group_body(g, accs):
            r0 = g * _G
            pv = par_v[row, pl.ds(r0, _G)]
            for t in range(_G):
                accs = acc_token(buf, r0 + t, pv, t, accs)
            return accs

        a = lax.fori_loop(0, _NGRP, group_body, (zero, zero, zero, zero))
        pv_tail = par_v[row, pl.ds(_NGRP * _G, _G)]
        for t in range(_TAIL):
            a = acc_token(buf, _NGRP * _G + t, pv_tail, t, a)

        t0 = a[0] * w[0][0] + a[1] * w[0][1] + a[2] * w[0][2] + a[3] * w[0][3]
        t1 = a[0] * w[1][0] + a[1] * w[1][1] + a[2] * w[1][2] + a[3] * w[1][3]
        l0 = jnp.sum(t0) + b0
        l1 = jnp.sum(t1) + b1
        slot = 2 * lax.rem(row, 8)
        lvec = jnp.where(lane == slot, l0, lvec)
        lvec = jnp.where(lane == slot + 1, l1, lvec)
        return lvec

    issue(0, buf_a, sem_a)

    def outer(i, lvec):
        issue(2 * i + 1, buf_b, sem_b)
        wait(buf_a, sem_a)
        lvec = process(2 * i, buf_a, lvec)

        @pl.when(i < _PAIRS - 1)
        def _():
            issue(2 * i + 2, buf_a, sem_a)

        wait(buf_b, sem_b)
        lvec = process(2 * i + 1, buf_b, lvec)

        @pl.when(lax.rem(i, 4) == 3)
        def _():
            log_v[pl.ds((i // 4) * _L, _L)] = lvec

        return lvec

    lax.fori_loop(0, _PAIRS, outer, jnp.zeros((_L,), jnp.float32))
    pltpu.sync_copy(log_v, out_hbm.at[pl.ds(base * _C, _BPW * _C)])


@jax.jit
def _sc_call(table2, idx3, par2, w_flat, b_pad):
    mesh = plsc.VectorSubcoreMesh(core_axis_name="c", subcore_axis_name="s",
                                  num_cores=_NC, num_subcores=_NS)
    return pl.kernel(
        _tec_body,
        out_type=jax.ShapeDtypeStruct((_B * _C,), jnp.float32),
        mesh=mesh,
        compiler_params=pltpu.CompilerParams(needs_layout_passes=False,
                                             use_tc_tiling_on_sc=True),
        scratch_types=[
            pltpu.VMEM((_BPW, 2, _CHUNK), jnp.int32),
            pltpu.VMEM((_BPW, _SPAD), jnp.int32),
            pltpu.VMEM((_S, 2 * _D), jnp.float32),
            pltpu.VMEM((_S, 2 * _D), jnp.float32),
            pltpu.VMEM((_C * _D,), jnp.float32),
            pltpu.VMEM((_L,), jnp.float32),
            pltpu.VMEM((_BPW * _C,), jnp.float32),
            pltpu.SemaphoreType.DMA,
            pltpu.SemaphoreType.DMA,
        ],
    )(table2, idx3, par2, w_flat, b_pad)


@jax.jit
def _tc_superrow_table(embedding):
    return pl.pallas_call(
        _transpose_body,
        grid=(_NTBLK,),
        in_specs=[pl.BlockSpec((_D, 2 * _TBLK), lambda g: (0, g))],
        out_specs=pl.BlockSpec((_TBLK, 2 * _D), lambda g: (g, 0)),
        out_shape=jax.ShapeDtypeStruct((_VSUP, 2 * _D), jnp.float32),
    )(embedding.T)


def kernel(x, embedding, W, b):
    xi = x.astype(jnp.int32)
    table2 = _tc_superrow_table(embedding)
    idx3 = ((xi // (2 * _TBLK)) * _TBLK + (xi % _TBLK)).reshape(_B, 2, _CHUNK)
    par2 = jnp.pad((xi // _TBLK) & 1, ((0, 0), (0, _SPAD - _S))).reshape(_B, _SPAD)
    w_flat = W.astype(jnp.float32).reshape(-1)
    b_pad = jnp.pad(b.astype(jnp.float32), (0, _L - _C))
    return _sc_call(table2, idx3, par2, w_flat, b_pad).reshape(_B, _C)


# speedup vs baseline: 1.9253x; 1.4504x over previous
import jax
import jax.numpy as jnp
from jax import lax
from jax.experimental import pallas as pl
from jax.experimental.pallas import tpu as pltpu
from jax.experimental.pallas import tpu_sc as plsc

_NC = 2
_NS = 16
_NW = _NC * _NS
_L = 16

_B = 4096
_S = 200
_D = 64
_C = 2
_BPW = _B // _NW
_CHUNK = _S // 2
_PAIRS = _BPW // 2
_V = 1000000
_TBLK = 4096
_NTBLK = (_V + 2 * _TBLK - 1) // (2 * _TBLK)
_VSUP = _NTBLK * _TBLK
_G = 16
_NGRP = _S // _G
_TAIL = _S - _NGRP * _G
_SPAD = _NGRP * _G + _G


def _transpose_body(in_ref, out_ref):
    x = in_ref[...]
    x2 = jnp.concatenate([x[:, 0:_TBLK], x[:, _TBLK:2 * _TBLK]], axis=0)
    out_ref[...] = jnp.transpose(x2, (1, 0))


def _tec_body(table, idx_hbm, par_hbm, w_hbm, b_hbm, out_hbm,
              idx_v, par_v, buf_a, buf_b, w_v, b_v, log_v, sem_a, sem_b):
    wid = lax.axis_index("s") * _NC + lax.axis_index("c")
    base = wid * _BPW

    pltpu.sync_copy(idx_hbm.at[pl.ds(base, _BPW)], idx_v)
    pltpu.sync_copy(par_hbm.at[pl.ds(base, _BPW)], par_v)
    pltpu.sync_copy(w_hbm, w_v)
    pltpu.sync_copy(b_hbm, b_v)

    inv_s = 1.0 / _S
    w = [[w_v[pl.ds(c * _D + k * _L, _L)] * inv_s for k in range(4)]
         for c in range(_C)]
    bvec = b_v[...]
    b0 = bvec[0]
    b1 = bvec[1]
    lane = jax.lax.iota(jnp.int32, _L)

    def issue(row, buf, sem):
        pltpu.async_copy(table.at[idx_v.at[row, 0]], buf.at[pl.ds(0, _CHUNK)], sem)
        pltpu.async_copy(table.at[idx_v.at[row, 1]], buf.at[pl.ds(_CHUNK, _CHUNK)], sem)

    def wait(buf, sem):
        pltpu.make_async_copy(table.at[pl.ds(0, _S)], buf, sem).wait()

    def acc_token(buf, r, pv, t, accs):
        sel = pv[t] > 0
        return tuple(
            accs[k]
            + jnp.where(sel,
                        buf[r, pl.ds(_D + k * _L, _L)],
                        buf[r, pl.ds(k * _L, _L)])
            for k in range(4)
        )

    def process(row, buf, lvec):
        zero = jnp.zeros((_L,), jnp.float32)

        def group_body(g, accs):
            r0 = g * _G
            pv = par_v[row, pl.ds(r0, _G)]
            for t in range(_G):
                accs = acc_token(buf, r0 + t, pv, t, accs)
            return accs

        a = lax.fori_loop(0, _NGRP, group_body, (zero, zero, zero, zero))
        pv_tail = par_v[row, pl.ds(_NGRP * _G, _G)]
        for t in range(_TAIL):
            a = acc_token(buf, _NGRP * _G + t, pv_tail, t, a)

        t0 = a[0] * w[0][0] + a[1] * w[0][1] + a[2] * w[0][2] + a[3] * w[0][3]
        t1 = a[0] * w[1][0] + a[1] * w[1][1] + a[2] * w[1][2] + a[3] * w[1][3]
        l0 = jnp.sum(t0) + b0
        l1 = jnp.sum(t1) + b1
        slot = 2 * lax.rem(row, 8)
        lvec = jnp.where(lane == slot, l0, lvec)
        lvec = jnp.where(lane == slot + 1, l1, lvec)
        return lvec

    issue(0, buf_a, sem_a)

    def outer(i, lvec):
        issue(2 * i + 1, buf_b, sem_b)
        wait(buf_a, sem_a)
        lvec = process(2 * i, buf_a, lvec)

        @pl.when(i < _PAIRS - 1)
        def _():
            issue(2 * i + 2, buf_a, sem_a)

        wait(buf_b, sem_b)
        lvec = process(2 * i + 1, buf_b, lvec)

        @pl.when(lax.rem(i, 4) == 3)
        def _():
            log_v[pl.ds((i // 4) * _L, _L)] = lvec

        return lvec

    lax.fori_loop(0, _PAIRS, outer, jnp.zeros((_L,), jnp.float32))
    pltpu.sync_copy(log_v, out_hbm.at[pl.ds(base * _C, _BPW * _C)])


@jax.jit
def _sc_call(table2, idx3, par2, w_flat, b_pad):
    mesh = plsc.VectorSubcoreMesh(core_axis_name="c", subcore_axis_name="s",
                                  num_cores=_NC, num_subcores=_NS)
    return pl.kernel(
        _tec_body,
        out_type=jax.ShapeDtypeStruct((_B * _C,), jnp.float32),
        mesh=mesh,
        compiler_params=pltpu.CompilerParams(needs_layout_passes=False,
                                             use_tc_tiling_on_sc=True),
        scratch_types=[
            pltpu.VMEM((_BPW, 2, _CHUNK), jnp.int32),
            pltpu.VMEM((_BPW, _SPAD), jnp.int32),
            pltpu.VMEM((_S, 2 * _D), jnp.float32),
            pltpu.VMEM((_S, 2 * _D), jnp.float32),
            pltpu.VMEM((_C * _D,), jnp.float32),
            pltpu.VMEM((_L,), jnp.float32),
            pltpu.VMEM((_BPW * _C,), jnp.float32),
            pltpu.SemaphoreType.DMA,
            pltpu.SemaphoreType.DMA,
        ],
    )(table2, idx3, par2, w_flat, b_pad)


@jax.jit
def _tc_superrow_table(embedding):
    return pl.pallas_call(
        _transpose_body,
        grid=(_NTBLK,),
        in_specs=[pl.BlockSpec((_D, 2 * _TBLK), lambda g: (0, g))],
        out_specs=pl.BlockSpec((_TBLK, 2 * _D), lambda g: (g, 0)),
        out_shape=jax.ShapeDtypeStruct((_VSUP, 2 * _D), jnp.float32),
    )(embedding.T)


def kernel(x, embedding, W, b):
    xi = x.astype(jnp.int32)
    table2 = _tc_superrow_table(embedding)
    idx3 = ((xi // (2 * _TBLK)) * _TBLK + (xi % _TBLK)).reshape(_B, 2, _CHUNK)
    par2 = jnp.pad((xi // _TBLK) & 1, ((0, 0), (0, _SPAD - _S))).reshape(_B, _SPAD)
    w_flat = W.astype(jnp.float32).reshape(-1)
    b_pad = jnp.pad(b.astype(jnp.float32), (0, _L - _C))
    return _sc_call(table2, idx3, par2, w_flat, b_pad).reshape(_B, _C)
